# baseline (speedup 1.0000x reference)
import functools

import jax
import jax.numpy as jnp
from jax import lax
from jax.experimental import pallas as pl
from jax.experimental.pallas import tpu as pltpu
from jax.experimental.pallas import tpu_sc as plsc

CH = 128
K = 80
NSUB = 16
NCORE = 2


def _lrelu(v):
    return jnp.where(v >= 0, v, 0.01 * v)


def _dot_t(a, b):
    return lax.dot_general(a, b, (((1,), (1,)), ((), ())),
                           preferred_element_type=jnp.float32)


def _edge_bias(edge_attr, W, b, nch, eb=2000):
    E, ed = edge_attr.shape
    nb = E // eb

    def body(ea, w, bb, out):
        c = pl.program_id(1)
        out[...] = _dot_t(ea[...], w[...]) + bb[pl.ds(c, 1), :]

    return pl.pallas_call(
        body, grid=(nb, nch),
        in_specs=[pl.BlockSpec((eb, ed), lambda i, c: (i, 0)),
                  pl.BlockSpec((CH, ed), lambda i, c: (c, 0)),
                  pl.BlockSpec((nch, CH), lambda i, c: (0, 0))],
        out_specs=pl.BlockSpec((eb, CH), lambda i, c: (c * nb + i, 0)),
        out_shape=jax.ShapeDtypeStruct((nch * E, CH), jnp.float32),
    )(edge_attr, W, b.reshape(nch, CH))


def _sc_gine_agg(src, dst, c_all, x_chunks, nch, N, E):
    KB = 80
    ZR = 40
    ept = E // NSUB
    nb = ept // KB
    zb = N // ZR
    mesh = plsc.VectorSubcoreMesh(core_axis_name="c", subcore_axis_name="s")
    out_type = tuple(jax.ShapeDtypeStruct((N, CH), jnp.float32)
                     for _ in range(nch))
    f32, i32 = jnp.float32, jnp.int32
    scratch = [pltpu.VMEM((KB,), i32), pltpu.VMEM((KB,), i32),
               pltpu.VMEM((KB,), i32), pltpu.VMEM((KB,), i32),
               pltpu.VMEM((KB, CH), f32), pltpu.VMEM((KB, CH), f32),
               pltpu.VMEM((KB, CH), f32), pltpu.VMEM((KB, CH), f32),
               pltpu.VMEM((ZR, CH), f32),
               pltpu.VMEM_SHARED((N, CH), f32),
               pltpu.SemaphoreType.DMA, pltpu.SemaphoreType.DMA,
               pltpu.SemaphoreType.DMA, pltpu.SemaphoreType.DMA,
               pltpu.SemaphoreType.DMA, pltpu.SemaphoreType.DMA]

    @functools.partial(pl.kernel, out_type=out_type, mesh=mesh,
                       scratch_types=scratch)
    def k(src_h, dst_h, c_h, *rest):
        xs = rest[:nch]
        outs = rest[nch:2 * nch]
        (ia0, ia1, dv0, dv1, g0, g1, c0, c1, zbuf, shared,
         sg0, sg1, si0, si1, ss0, ss1) = rest[2 * nch:]
        ia = (ia0, ia1)
        dv = (dv0, dv1)
        gb = (g0, g1)
        cb = (c0, c1)
        sg = (sg0, sg1)
        si = (si0, si1)
        ss = (ss0, ss1)
        cid = lax.axis_index("c")
        sid = lax.axis_index("s")

        def zrow(r, carry):
            for q in range(CH // 16):
                zbuf[r, pl.ds(q * 16, 16)] = jnp.zeros((16,), f32)
            return carry
        lax.fori_loop(0, ZR, zrow, 0)

        def issue_io(b, p, chunk):
            e0 = sid * ept + b * KB
            pltpu.async_copy(src_h.at[pl.ds(e0, KB)], ia[p], si[p])
            pltpu.async_copy(dst_h.at[pl.ds(e0, KB)], dv[p], si[p])
            pltpu.async_copy(c_h.at[pl.ds(chunk * E + e0, KB), :],
                             cb[p], si[p])

        def drain_io(p):
            pltpu.make_async_copy(src_h.at[pl.ds(0, KB)], ia[p],
                                  si[p]).wait()
            pltpu.make_async_copy(dst_h.at[pl.ds(0, KB)], dv[p],
                                  si[p]).wait()
            pltpu.make_async_copy(c_h.at[pl.ds(0, KB), :], cb[p],
                                  si[p]).wait()

        def issue_gather(p, chunk):
            pltpu.async_copy(xs[chunk].at[ia[p]], gb[p], sg[p])

        def drain_gather(p, chunk):
            pltpu.make_async_copy(xs[chunk].at[ia[p]], gb[p],
                                  sg[p]).wait()

        for chunk in range(nch):
            @pl.when(cid == (chunk % NCORE))
            def _(chunk=chunk):
                nz_full, rem = divmod(zb, NSUB)
                for z in range(nz_full):
                    blk = sid + NSUB * z
                    pltpu.sync_copy(zbuf, shared.at[pl.ds(blk * ZR, ZR), :])
                if rem:
                    @pl.when(sid < rem)
                    def _():
                        blk = sid + NSUB * nz_full
                        pltpu.sync_copy(zbuf,
                                        shared.at[pl.ds(blk * ZR, ZR), :])
                plsc.subcore_barrier()

                e0 = sid * ept
                pltpu.sync_copy(src_h.at[pl.ds(e0, KB)], ia[0])
                pltpu.sync_copy(dst_h.at[pl.ds(e0, KB)], dv[0])
                pltpu.sync_copy(c_h.at[pl.ds(chunk * E + e0, KB), :], cb[0])
                issue_gather(0, chunk)
                issue_io(1, 1, chunk)

                def scat_desc(p):
                    return pltpu.make_async_copy(gb[p], shared.at[dv[p]],
                                                 ss[p])

                def step(b, p):
                    @pl.when(b + 1 <= nb - 1)
                    def _():
                        drain_io(1 - p)
                    drain_gather(p, chunk)

                    @plsc.parallel_loop(0, KB, 1, unroll=4)
                    def _(r):
                        for q in range(CH // 16):
                            s = pl.ds(q * 16, 16)
                            gb[p][r, s] = jnp.maximum(
                                gb[p][r, s] + cb[p][r, s], 0.0)

                    @pl.when(b + 2 <= nb - 1)
                    def _():
                        issue_io(b + 2, p, chunk)
                    @pl.when(b >= 1)
                    def _():
                        scat_desc(1 - p).wait()
                    @pl.when(b + 1 <= nb - 1)
                    def _():
                        issue_gather(1 - p, chunk)
                    scat_desc(p).start(add=True)

                def pair(j, carry):
                    step(2 * j, 0)
                    step(2 * j + 1, 1)
                    return carry
                lax.fori_loop(0, nb // 2, pair, 0)
                if nb % 2:
                    step(nb - 1, 0)
                scat_desc((nb - 1) % 2).wait()
                plsc.subcore_barrier()

                nz_full, rem = divmod(zb, NSUB)
                for z in range(nz_full):
                    blk = sid + NSUB * z
                    pltpu.sync_copy(shared.at[pl.ds(blk * ZR, ZR), :],
                                    outs[chunk].at[pl.ds(blk * ZR, ZR), :])
                if rem:
                    @pl.when(sid < rem)
                    def _():
                        blk = sid + NSUB * nz_full
                        pltpu.sync_copy(
                            shared.at[pl.ds(blk * ZR, ZR), :],
                            outs[chunk].at[pl.ds(blk * ZR, ZR), :])

    return k(src, dst, c_all, *x_chunks)


def _gin_mlp(self_chunks, agg_chunks, W1, b1, W2, b2, relu_out, out_chunked):
    N = self_chunks[0].shape[0]
    nin = len(self_chunks)
    Hh = W1.shape[0]
    nbl = 10
    Nb = N // nbl
    nco = Hh // CH

    def body(*refs):
        ins = refs[:nin]
        aggs = refs[nin:2 * nin]
        w1, bb1, w2, bb2 = refs[2 * nin:2 * nin + 4]
        outs = refs[2 * nin + 4:]
        h0 = jnp.concatenate(
            [ins[i][...] + aggs[i][...] for i in range(nin)], axis=1)
        t = _lrelu(_dot_t(h0, w1[...]) + bb1[...])
        u = _lrelu(_dot_t(t, w2[...]) + bb2[...])
        if relu_out:
            u = jnp.maximum(u, 0.0)
        if out_chunked:
            for q in range(nco):
                outs[q][...] = u[:, q * CH:(q + 1) * CH]
        else:
            outs[0][...] = u

    in_specs = ([pl.BlockSpec((Nb, CH), lambda i: (i, 0))] * (2 * nin)
                + [pl.BlockSpec(W1.shape, lambda i: (0, 0)),
                   pl.BlockSpec((1, Hh), lambda i: (0, 0)),
                   pl.BlockSpec(W2.shape, lambda i: (0, 0)),
                   pl.BlockSpec((1, Hh), lambda i: (0, 0))])
    if out_chunked:
        out_shape = tuple(jax.ShapeDtypeStruct((N, CH), jnp.float32)
                          for _ in range(nco))
        out_specs = tuple(pl.BlockSpec((Nb, CH), lambda i: (i, 0))
                          for _ in range(nco))
    else:
        out_shape = jax.ShapeDtypeStruct((N, Hh), jnp.float32)
        out_specs = pl.BlockSpec((Nb, Hh), lambda i: (i, 0))
    return pl.pallas_call(
        body, grid=(nbl,), in_specs=in_specs, out_specs=out_specs,
        out_shape=out_shape,
    )(*self_chunks, *agg_chunks, W1, b1.reshape(1, Hh), W2, b2.reshape(1, Hh))


def _head(h2, target_object, Wa1, ba1, Wa2, ba2):
    Bb, S = target_object.shape
    Hh = Wa1.shape[0]
    A = Wa2.shape[0]

    def body(to, hb, w1, bb1, w2, bb2, out):
        h3 = hb[...].reshape(Bb, S, Hh)
        sel = lax.dot_general(to[...], h3, (((1,), (1,)), ((0,), (0,))),
                              preferred_element_type=jnp.float32)
        a = _lrelu(_dot_t(sel, w1[...]) + bb1[...])
        a = _lrelu(_dot_t(a, w2[...]) + bb2[...])
        m = jnp.max(a, axis=1, keepdims=True)
        e = jnp.exp(a - m)
        out[...] = e / jnp.sum(e, axis=1, keepdims=True)

    return pl.pallas_call(
        body, grid=(1,),
        in_specs=[pl.BlockSpec((Bb, S), lambda i: (0, 0)),
                  pl.BlockSpec((Bb * S, Hh), lambda i: (0, 0)),
                  pl.BlockSpec((Hh, Hh), lambda i: (0, 0)),
                  pl.BlockSpec((1, Hh), lambda i: (0, 0)),
                  pl.BlockSpec((A, Hh), lambda i: (0, 0)),
                  pl.BlockSpec((1, A), lambda i: (0, 0))],
        out_specs=pl.BlockSpec((Bb, A), lambda i: (0, 0)),
        out_shape=jax.ShapeDtypeStruct((Bb, A), jnp.float32),
    )(target_object, h2, Wa1, ba1.reshape(1, Hh), Wa2, ba2.reshape(1, A))


def kernel(x, edge_index, edge_attr, target_object, W_e1, b_e1, W11, b11,
           W12, b12, W_e2, b_e2, W21, b21, W22, b22, Wa1, ba1, Wa2, ba2):
    N, D = x.shape
    E = edge_attr.shape[0]
    Hh = W11.shape[0]
    src = edge_index[0]
    dst = edge_index[1]
    x_chunks = [x[:, i * CH:(i + 1) * CH] for i in range(D // CH)]

    c1 = _edge_bias(edge_attr, W_e1, b_e1, D // CH)
    c2 = _edge_bias(edge_attr, W_e2, b_e2, Hh // CH)
    agg1 = _sc_gine_agg(src, dst, c1, x_chunks, D // CH, N, E)
    h1 = _gin_mlp(x_chunks, list(agg1), W11, b11, W12, b12,
                  relu_out=True, out_chunked=True)
    agg2 = _sc_gine_agg(src, dst, c2, list(h1), Hh // CH, N, E)
    h2 = _gin_mlp(list(h1), list(agg2), W21, b21, W22, b22,
                  relu_out=False, out_chunked=False)
    return _head(h2, target_object, Wa1, ba1, Wa2, ba2)


# speedup vs baseline: 1.0015x; 1.0015x over previous
import functools

import jax
import jax.numpy as jnp
from jax import lax
from jax.experimental import pallas as pl
from jax.experimental.pallas import tpu as pltpu
from jax.experimental.pallas import tpu_sc as plsc

CH = 128
KB = 80
ZR = 40
NSUB = 16
NCORE = 2


def _lrelu(v):
    return jnp.where(v >= 0, v, 0.01 * v)


def _dot_t(a, b):
    return lax.dot_general(a, b, (((1,), (1,)), ((), ())),
                           preferred_element_type=jnp.float32)


def _edge_bias(edge_attr, W, b, nch, eb=2000):
    E, ed = edge_attr.shape
    nb = E // eb

    def body(ea, w, bb, out):
        c = pl.program_id(1)
        out[...] = _dot_t(ea[...], w[...]) + bb[pl.ds(c, 1), :]

    return pl.pallas_call(
        body, grid=(nb, nch),
        in_specs=[pl.BlockSpec((eb, ed), lambda i, c: (i, 0)),
                  pl.BlockSpec((CH, ed), lambda i, c: (c, 0)),
                  pl.BlockSpec((nch, CH), lambda i, c: (0, 0))],
        out_specs=pl.BlockSpec((eb, CH), lambda i, c: (c * nb + i, 0)),
        out_shape=jax.ShapeDtypeStruct((nch * E, CH), jnp.float32),
    )(edge_attr, W, b.reshape(nch, CH))


def _sc_gine_agg(src, dst, c_all, x_chunks, nch, N, E):
    ept = E // NSUB
    nb = ept // KB
    zb = N // ZR
    mesh = plsc.VectorSubcoreMesh(core_axis_name="c", subcore_axis_name="s")
    out_type = tuple(jax.ShapeDtypeStruct((N, CH), jnp.float32)
                     for _ in range(nch))
    f32, i32 = jnp.float32, jnp.int32
    scratch = [pltpu.VMEM((KB,), i32), pltpu.VMEM((KB,), i32),
               pltpu.VMEM((KB,), i32), pltpu.VMEM((KB,), i32),
               pltpu.VMEM((KB, CH), f32), pltpu.VMEM((KB, CH), f32),
               pltpu.VMEM((KB, CH), f32), pltpu.VMEM((KB, CH), f32),
               pltpu.VMEM((ZR, CH), f32),
               pltpu.VMEM_SHARED((N, CH), f32),
               pltpu.SemaphoreType.DMA, pltpu.SemaphoreType.DMA,
               pltpu.SemaphoreType.DMA, pltpu.SemaphoreType.DMA,
               pltpu.SemaphoreType.DMA, pltpu.SemaphoreType.DMA,
               pltpu.SemaphoreType.DMA, pltpu.SemaphoreType.DMA]

    @functools.partial(pl.kernel, out_type=out_type, mesh=mesh,
                       scratch_types=scratch)
    def k(src_h, dst_h, c_h, *rest):
        xs = rest[:nch]
        outs = rest[nch:2 * nch]
        (ia0, ia1, dv0, dv1, g0, g1, c0, c1, zbuf, shared,
         sg0, sg1, si0, si1, sd0, sd1, ss0, ss1) = rest[2 * nch:]
        ia = (ia0, ia1)
        dv = (dv0, dv1)
        gb = (g0, g1)
        cb = (c0, c1)
        sg = (sg0, sg1)
        si = (si0, si1)
        sd = (sd0, sd1)
        ss = (ss0, ss1)
        cid = lax.axis_index("c")
        sid = lax.axis_index("s")

        def zrow(r, carry):
            for q in range(CH // 16):
                zbuf[r, pl.ds(q * 16, 16)] = jnp.zeros((16,), f32)
            return carry
        lax.fori_loop(0, ZR, zrow, 0)

        def issue_io(b, p, chunk):
            e0 = pl.multiple_of(sid * ept + b * KB, KB)
            eh = pl.multiple_of(chunk * E + e0, KB)
            pltpu.async_copy(src_h.at[pl.ds(e0, KB)], ia[p], si[p])
            pltpu.async_copy(c_h.at[pl.ds(eh, KB), :], cb[p], si[p])

        def drain_io(p):
            pltpu.make_async_copy(src_h.at[pl.ds(0, KB)], ia[p],
                                  si[p]).wait()
            pltpu.make_async_copy(c_h.at[pl.ds(0, KB), :], cb[p],
                                  si[p]).wait()

        def issue_dv(b, p):
            e0 = pl.multiple_of(sid * ept + b * KB, KB)
            pltpu.async_copy(dst_h.at[pl.ds(e0, KB)], dv[p], sd[p])

        def drain_dv(p):
            pltpu.make_async_copy(dst_h.at[pl.ds(0, KB)], dv[p],
                                  sd[p]).wait()

        def issue_gather(p, chunk):
            pltpu.async_copy(xs[chunk].at[ia[p]], gb[p], sg[p])

        def drain_gather(p, chunk):
            pltpu.make_async_copy(xs[chunk].at[ia[p]], gb[p],
                                  sg[p]).wait()

        for chunk in range(nch):
            @pl.when(cid == (chunk % NCORE))
            def _(chunk=chunk):
                nz_full, rem = divmod(zb, NSUB)
                for z in range(nz_full):
                    blk = sid + NSUB * z
                    pltpu.sync_copy(zbuf, shared.at[pl.ds(blk * ZR, ZR), :])
                if rem:
                    @pl.when(sid < rem)
                    def _():
                        blk = sid + NSUB * nz_full
                        pltpu.sync_copy(zbuf,
                                        shared.at[pl.ds(blk * ZR, ZR), :])
                plsc.subcore_barrier()

                e0 = pl.multiple_of(sid * ept, KB)
                eh = pl.multiple_of(chunk * E + e0, KB)
                pltpu.sync_copy(src_h.at[pl.ds(e0, KB)], ia[0])
                pltpu.sync_copy(c_h.at[pl.ds(eh, KB), :], cb[0])
                issue_dv(0, 0)
                issue_gather(0, chunk)
                issue_io(1, 1, chunk)

                def scat_desc(p):
                    return pltpu.make_async_copy(gb[p], shared.at[dv[p]],
                                                 ss[p])

                def step(b, p):
                    @pl.when(b + 1 <= nb - 1)
                    def _():
                        drain_io(1 - p)
                    drain_gather(p, chunk)

                    @plsc.parallel_loop(0, KB, 1, unroll=4)
                    def _(r):
                        for q in range(CH // 16):
                            s = pl.ds(q * 16, 16)
                            gb[p][r, s] = jnp.maximum(
                                gb[p][r, s] + cb[p][r, s], 0.0)

                    @pl.when(b + 2 <= nb - 1)
                    def _():
                        issue_io(b + 2, p, chunk)
                    @pl.when(b >= 1)
                    def _():
                        scat_desc(1 - p).wait()
                    @pl.when(b + 1 <= nb - 1)
                    def _():
                        issue_gather(1 - p, chunk)
                        issue_dv(b + 1, 1 - p)
                    drain_dv(p)
                    scat_desc(p).start(add=True)

                def pair(j, carry):
                    step(2 * j, 0)
                    step(2 * j + 1, 1)
                    return carry
                lax.fori_loop(0, nb // 2, pair, 0)
                if nb % 2:
                    step(nb - 1, 0)
                scat_desc((nb - 1) % 2).wait()
                plsc.subcore_barrier()

                nz_full, rem = divmod(zb, NSUB)
                for z in range(nz_full):
                    blk = sid + NSUB * z
                    pltpu.sync_copy(shared.at[pl.ds(blk * ZR, ZR), :],
                                    outs[chunk].at[pl.ds(blk * ZR, ZR), :])
                if rem:
                    @pl.when(sid < rem)
                    def _():
                        blk = sid + NSUB * nz_full
                        pltpu.sync_copy(
                            shared.at[pl.ds(blk * ZR, ZR), :],
                            outs[chunk].at[pl.ds(blk * ZR, ZR), :])

    return k(src, dst, c_all, *x_chunks)


def _gin_mlp(self_chunks, agg_chunks, W1, b1, W2, b2, relu_out, out_chunked):
    N = self_chunks[0].shape[0]
    nin = len(self_chunks)
    Hh = W1.shape[0]
    nbl = 10
    Nb = N // nbl
    nco = Hh // CH

    def body(*refs):
        ins = refs[:nin]
        aggs = refs[nin:2 * nin]
        w1, bb1, w2, bb2 = refs[2 * nin:2 * nin + 4]
        outs = refs[2 * nin + 4:]
        h0 = jnp.concatenate(
            [ins[i][...] + aggs[i][...] for i in range(nin)], axis=1)
        t = _lrelu(_dot_t(h0, w1[...]) + bb1[...])
        u = _lrelu(_dot_t(t, w2[...]) + bb2[...])
        if relu_out:
            u = jnp.maximum(u, 0.0)
        if out_chunked:
            for q in range(nco):
                outs[q][...] = u[:, q * CH:(q + 1) * CH]
        else:
            outs[0][...] = u

    in_specs = ([pl.BlockSpec((Nb, CH), lambda i: (i, 0))] * (2 * nin)
                + [pl.BlockSpec(W1.shape, lambda i: (0, 0)),
                   pl.BlockSpec((1, Hh), lambda i: (0, 0)),
                   pl.BlockSpec(W2.shape, lambda i: (0, 0)),
                   pl.BlockSpec((1, Hh), lambda i: (0, 0))])
    if out_chunked:
        out_shape = tuple(jax.ShapeDtypeStruct((N, CH), jnp.float32)
                          for _ in range(nco))
        out_specs = tuple(pl.BlockSpec((Nb, CH), lambda i: (i, 0))
                          for _ in range(nco))
    else:
        out_shape = jax.ShapeDtypeStruct((N, Hh), jnp.float32)
        out_specs = pl.BlockSpec((Nb, Hh), lambda i: (i, 0))
    return pl.pallas_call(
        body, grid=(nbl,), in_specs=in_specs, out_specs=out_specs,
        out_shape=out_shape,
    )(*self_chunks, *agg_chunks, W1, b1.reshape(1, Hh), W2,
      b2.reshape(1, Hh))


def _head(h2, target_object, Wa1, ba1, Wa2, ba2):
    Bb, S = target_object.shape
    Hh = Wa1.shape[0]
    A = Wa2.shape[0]

    def body(to, hb, w1, bb1, w2, bb2, out):
        h3 = hb[...].reshape(Bb, S, Hh)
        sel = lax.dot_general(to[...], h3, (((1,), (1,)), ((0,), (0,))),
                              preferred_element_type=jnp.float32)
        a = _lrelu(_dot_t(sel, w1[...]) + bb1[...])
        a = _lrelu(_dot_t(a, w2[...]) + bb2[...])
        m = jnp.max(a, axis=1, keepdims=True)
        e = jnp.exp(a - m)
        out[...] = e / jnp.sum(e, axis=1, keepdims=True)

    return pl.pallas_call(
        body, grid=(1,),
        in_specs=[pl.BlockSpec((Bb, S), lambda i: (0, 0)),
                  pl.BlockSpec((Bb * S, Hh), lambda i: (0, 0)),
                  pl.BlockSpec((Hh, Hh), lambda i: (0, 0)),
                  pl.BlockSpec((1, Hh), lambda i: (0, 0)),
                  pl.BlockSpec((A, Hh), lambda i: (0, 0)),
                  pl.BlockSpec((1, A), lambda i: (0, 0))],
        out_specs=pl.BlockSpec((Bb, A), lambda i: (0, 0)),
        out_shape=jax.ShapeDtypeStruct((Bb, A), jnp.float32),
    )(target_object, h2, Wa1, ba1.reshape(1, Hh), Wa2, ba2.reshape(1, A))


def kernel(x, edge_index, edge_attr, target_object, W_e1, b_e1, W11, b11,
           W12, b12, W_e2, b_e2, W21, b21, W22, b22, Wa1, ba1, Wa2, ba2):
    N, D = x.shape
    E = edge_attr.shape[0]
    Hh = W11.shape[0]
    src = edge_index[0]
    dst = edge_index[1]

    x_chunks = [x[:, i * CH:(i + 1) * CH] for i in range(D // CH)]

    c1 = _edge_bias(edge_attr, W_e1, b_e1, D // CH)
    c2 = _edge_bias(edge_attr, W_e2, b_e2, Hh // CH)

    agg1 = _sc_gine_agg(src, dst, c1, x_chunks, D // CH, N, E)
    h1 = _gin_mlp(x_chunks, list(agg1), W11, b11, W12, b12,
                  relu_out=True, out_chunked=True)
    agg2 = _sc_gine_agg(src, dst, c2, list(h1), Hh // CH, N, E)
    h2 = _gin_mlp(list(h1), list(agg2), W21, b21, W22, b22,
                  relu_out=False, out_chunked=False)
    return _head(h2, target_object, Wa1, ba1, Wa2, ba2)


# speedup vs baseline: 1.0025x; 1.0011x over previous
import functools

import jax
import jax.numpy as jnp
from jax import lax
from jax.experimental import pallas as pl
from jax.experimental.pallas import tpu as pltpu
from jax.experimental.pallas import tpu_sc as plsc

CH = 128
KB = 80
ZR = 40
NSUB = 16
NCORE = 2


def _lrelu(v):
    return jnp.where(v >= 0, v, 0.01 * v)


def _dot_t(a, b):
    return lax.dot_general(a, b, (((1,), (1,)), ((), ())),
                           preferred_element_type=jnp.float32)


def _edge_bias(edge_attr, W, b, nch, eb=2000):
    E, ed = edge_attr.shape
    nb = E // eb

    def body(ea, w, bb, out):
        c = pl.program_id(1)
        out[...] = _dot_t(ea[...], w[...]) + bb[pl.ds(c, 1), :]

    return pl.pallas_call(
        body, grid=(nb, nch),
        in_specs=[pl.BlockSpec((eb, ed), lambda i, c: (i, 0)),
                  pl.BlockSpec((CH, ed), lambda i, c: (c, 0)),
                  pl.BlockSpec((nch, CH), lambda i, c: (0, 0))],
        out_specs=pl.BlockSpec((eb, CH), lambda i, c: (c * nb + i, 0)),
        out_shape=jax.ShapeDtypeStruct((nch * E, CH), jnp.float32),
    )(edge_attr, W, b.reshape(nch, CH))


def _sc_gine_agg(src, dst, c_all, x_chunks, nch, N, E):
    ept = E // NSUB
    nb = ept // KB
    zb = N // ZR
    mesh = plsc.VectorSubcoreMesh(core_axis_name="c", subcore_axis_name="s")
    out_type = tuple(jax.ShapeDtypeStruct((N, CH), jnp.float32)
                     for _ in range(nch))
    f32, i32 = jnp.float32, jnp.int32
    scratch = [pltpu.VMEM((KB,), i32), pltpu.VMEM((KB,), i32),
               pltpu.VMEM((KB,), i32), pltpu.VMEM((KB,), i32),
               pltpu.VMEM((KB, CH), f32), pltpu.VMEM((KB, CH), f32),
               pltpu.VMEM((KB, CH), f32), pltpu.VMEM((KB, CH), f32),
               pltpu.VMEM((ZR, CH), f32),
               pltpu.VMEM_SHARED((N, CH), f32),
               pltpu.SemaphoreType.DMA, pltpu.SemaphoreType.DMA,
               pltpu.SemaphoreType.DMA, pltpu.SemaphoreType.DMA,
               pltpu.SemaphoreType.DMA, pltpu.SemaphoreType.DMA,
               pltpu.SemaphoreType.DMA, pltpu.SemaphoreType.DMA]

    @functools.partial(pl.kernel, out_type=out_type, mesh=mesh,
                       scratch_types=scratch)
    def k(src_h, dst_h, c_h, *rest):
        xs = rest[:nch]
        outs = rest[nch:2 * nch]
        (ia0, ia1, dv0, dv1, g0, g1, c0, c1, zbuf, shared,
         sg0, sg1, si0, si1, sd0, sd1, ss0, ss1) = rest[2 * nch:]
        ia = (ia0, ia1)
        dv = (dv0, dv1)
        gb = (g0, g1)
        cb = (c0, c1)
        sg = (sg0, sg1)
        si = (si0, si1)
        sd = (sd0, sd1)
        ss = (ss0, ss1)
        cid = lax.axis_index("c")
        sid = lax.axis_index("s")

        def zrow(r, carry):
            for q in range(CH // 16):
                zbuf[r, pl.ds(q * 16, 16)] = jnp.zeros((16,), f32)
            return carry
        lax.fori_loop(0, ZR, zrow, 0)

        def issue_io(b, p, chunk):
            e0 = pl.multiple_of(sid * ept + b * KB, KB)
            eh = pl.multiple_of(chunk * E + e0, KB)
            pltpu.async_copy(src_h.at[pl.ds(e0, KB)], ia[p], si[p])
            pltpu.async_copy(c_h.at[pl.ds(eh, KB), :], cb[p], si[p])

        def drain_io(p):
            pltpu.make_async_copy(src_h.at[pl.ds(0, KB)], ia[p],
                                  si[p]).wait()
            pltpu.make_async_copy(c_h.at[pl.ds(0, KB), :], cb[p],
                                  si[p]).wait()

        def issue_dv(b, p):
            e0 = pl.multiple_of(sid * ept + b * KB, KB)
            pltpu.async_copy(dst_h.at[pl.ds(e0, KB)], dv[p], sd[p])

        def drain_dv(p):
            pltpu.make_async_copy(dst_h.at[pl.ds(0, KB)], dv[p],
                                  sd[p]).wait()

        def issue_gather(p, chunk):
            pltpu.async_copy(xs[chunk].at[ia[p]], gb[p], sg[p])

        def drain_gather(p, chunk):
            pltpu.make_async_copy(xs[chunk].at[ia[p]], gb[p],
                                  sg[p]).wait()

        for chunk in range(nch):
            @pl.when(cid == (chunk % NCORE))
            def _(chunk=chunk):
                nz_full, rem = divmod(zb, NSUB)
                for z in range(nz_full):
                    blk = sid + NSUB * z
                    pltpu.sync_copy(zbuf, shared.at[pl.ds(blk * ZR, ZR), :])
                if rem:
                    @pl.when(sid < rem)
                    def _():
                        blk = sid + NSUB * nz_full
                        pltpu.sync_copy(zbuf,
                                        shared.at[pl.ds(blk * ZR, ZR), :])
                plsc.subcore_barrier()

                e0 = pl.multiple_of(sid * ept, KB)
                eh = pl.multiple_of(chunk * E + e0, KB)
                pltpu.sync_copy(src_h.at[pl.ds(e0, KB)], ia[0])
                pltpu.sync_copy(c_h.at[pl.ds(eh, KB), :], cb[0])
                issue_dv(0, 0)
                issue_gather(0, chunk)
                issue_io(1, 1, chunk)

                def scat_desc(p):
                    return pltpu.make_async_copy(gb[p], shared.at[dv[p]],
                                                 ss[p])

                def step(b, p):
                    @pl.when(b + 1 <= nb - 1)
                    def _():
                        drain_io(1 - p)
                    drain_gather(p, chunk)

                    @plsc.parallel_loop(0, KB, 1, unroll=4)
                    def _(r):
                        for q in range(CH // 16):
                            s = pl.ds(q * 16, 16)
                            gb[p][r, s] = jnp.maximum(
                                gb[p][r, s] + cb[p][r, s], 0.0)

                    @pl.when(b + 2 <= nb - 1)
                    def _():
                        issue_io(b + 2, p, chunk)
                    @pl.when(b >= 1)
                    def _():
                        scat_desc(1 - p).wait()
                    @pl.when(b + 1 <= nb - 1)
                    def _():
                        issue_gather(1 - p, chunk)
                        issue_dv(b + 1, 1 - p)
                    drain_dv(p)
                    scat_desc(p).start(add=True)

                def pair(j, carry):
                    step(2 * j, 0)
                    step(2 * j + 1, 1)
                    return carry
                lax.fori_loop(0, nb // 2, pair, 0)
                if nb % 2:
                    step(nb - 1, 0)
                scat_desc((nb - 1) % 2).wait()
                plsc.subcore_barrier()

                nz_full, rem = divmod(zb, NSUB)
                for z in range(nz_full):
                    blk = sid + NSUB * z
                    pltpu.sync_copy(shared.at[pl.ds(blk * ZR, ZR), :],
                                    outs[chunk].at[pl.ds(blk * ZR, ZR), :])
                if rem:
                    @pl.when(sid < rem)
                    def _():
                        blk = sid + NSUB * nz_full
                        pltpu.sync_copy(
                            shared.at[pl.ds(blk * ZR, ZR), :],
                            outs[chunk].at[pl.ds(blk * ZR, ZR), :])

    return k(src, dst, c_all, *x_chunks)


def _gin_mlp(self_chunks, agg_chunks, W1, b1, W2, b2, relu_out, out_chunked):
    N = self_chunks[0].shape[0]
    nin = len(self_chunks)
    Hh = W1.shape[0]
    nbl = 10
    Nb = N // nbl
    nco = Hh // CH

    def body(*refs):
        ins = refs[:nin]
        aggs = refs[nin:2 * nin]
        w1, bb1, w2, bb2 = refs[2 * nin:2 * nin + 4]
        outs = refs[2 * nin + 4:]
        h0 = jnp.concatenate(
            [ins[i][...] + aggs[i][...] for i in range(nin)], axis=1)
        t = _lrelu(_dot_t(h0, w1[...]) + bb1[...])
        u = _lrelu(_dot_t(t, w2[...]) + bb2[...])
        if relu_out:
            u = jnp.maximum(u, 0.0)
        if out_chunked:
            for q in range(nco):
                outs[q][...] = u[:, q * CH:(q + 1) * CH]
        else:
            outs[0][...] = u

    in_specs = ([pl.BlockSpec((Nb, CH), lambda i: (i, 0))] * (2 * nin)
                + [pl.BlockSpec(W1.shape, lambda i: (0, 0)),
                   pl.BlockSpec((1, Hh), lambda i: (0, 0)),
                   pl.BlockSpec(W2.shape, lambda i: (0, 0)),
                   pl.BlockSpec((1, Hh), lambda i: (0, 0))])
    if out_chunked:
        out_shape = tuple(jax.ShapeDtypeStruct((N, CH), jnp.float32)
                          for _ in range(nco))
        out_specs = tuple(pl.BlockSpec((Nb, CH), lambda i: (i, 0))
                          for _ in range(nco))
    else:
        out_shape = jax.ShapeDtypeStruct((N, Hh), jnp.float32)
        out_specs = pl.BlockSpec((Nb, Hh), lambda i: (i, 0))
    return pl.pallas_call(
        body, grid=(nbl,), in_specs=in_specs, out_specs=out_specs,
        out_shape=out_shape,
    )(*self_chunks, *agg_chunks, W1, b1.reshape(1, Hh), W2,
      b2.reshape(1, Hh))


def _head(h2, target_object, Wa1, ba1, Wa2, ba2):
    Bb, S = target_object.shape
    Hh = Wa1.shape[0]
    A = Wa2.shape[0]

    def body(to, hb, w1, bb1, w2, bb2, out):
        h3 = hb[...].reshape(Bb, S, Hh)
        sel = lax.dot_general(to[...], h3, (((1,), (1,)), ((0,), (0,))),
                              preferred_element_type=jnp.float32)
        a = _lrelu(_dot_t(sel, w1[...]) + bb1[...])
        a = _lrelu(_dot_t(a, w2[...]) + bb2[...])
        m = jnp.max(a, axis=1, keepdims=True)
        e = jnp.exp(a - m)
        out[...] = e / jnp.sum(e, axis=1, keepdims=True)

    return pl.pallas_call(
        body, grid=(1,),
        in_specs=[pl.BlockSpec((Bb, S), lambda i: (0, 0)),
                  pl.BlockSpec((Bb * S, Hh), lambda i: (0, 0)),
                  pl.BlockSpec((Hh, Hh), lambda i: (0, 0)),
                  pl.BlockSpec((1, Hh), lambda i: (0, 0)),
                  pl.BlockSpec((A, Hh), lambda i: (0, 0)),
                  pl.BlockSpec((1, A), lambda i: (0, 0))],
        out_specs=pl.BlockSpec((Bb, A), lambda i: (0, 0)),
        out_shape=jax.ShapeDtypeStruct((Bb, A), jnp.float32),
    )(target_object, h2, Wa1, ba1.reshape(1, Hh), Wa2, ba2.reshape(1, A))


def kernel(x, edge_index, edge_attr, target_object, W_e1, b_e1, W11, b11,
           W12, b12, W_e2, b_e2, W21, b21, W22, b22, Wa1, ba1, Wa2, ba2):
    N, D = x.shape
    E = edge_attr.shape[0]
    Hh = W11.shape[0]
    src = edge_index[0]
    dst = edge_index[1]

    x_chunks = [x[:, i * CH:(i + 1) * CH] for i in range(D // CH)]

    c1 = _edge_bias(edge_attr, W_e1, b_e1, D // CH)

    agg1 = _sc_gine_agg(src, dst, c1, x_chunks, D // CH, N, E)
    c2 = _edge_bias(edge_attr, W_e2, b_e2, Hh // CH)
    h1 = _gin_mlp(x_chunks, list(agg1), W11, b11, W12, b12,
                  relu_out=True, out_chunked=True)
    agg2 = _sc_gine_agg(src, dst, c2, list(h1), Hh // CH, N, E)
    h2 = _gin_mlp(list(h1), list(agg2), W21, b21, W22, b22,
                  relu_out=False, out_chunked=False)
    return _head(h2, target_object, Wa1, ba1, Wa2, ba2)


# speedup vs baseline: 1.0133x; 1.0107x over previous
import functools

import jax
import jax.numpy as jnp
from jax import lax
from jax.experimental import pallas as pl
from jax.experimental.pallas import tpu as pltpu
from jax.experimental.pallas import tpu_sc as plsc

CH = 128
KB = 80
ZR = 40
NSUB = 16
NCORE = 2


def _lrelu(v):
    return jnp.where(v >= 0, v, 0.01 * v)


def _dot_t(a, b):
    return lax.dot_general(a, b, (((1,), (1,)), ((), ())),
                           preferred_element_type=jnp.float32)


def _edge_bias(edge_attr, W, b, nch, eb=2000):
    E, ed = edge_attr.shape
    nb = E // eb

    def body(ea, w, bb, out):
        c = pl.program_id(1)
        out[...] = _dot_t(ea[...], w[...]) + bb[pl.ds(c, 1), :]

    return pl.pallas_call(
        body, grid=(nb, nch),
        in_specs=[pl.BlockSpec((eb, ed), lambda i, c: (i, 0)),
                  pl.BlockSpec((CH, ed), lambda i, c: (c, 0)),
                  pl.BlockSpec((nch, CH), lambda i, c: (0, 0))],
        out_specs=pl.BlockSpec((eb, CH), lambda i, c: (c * nb + i, 0)),
        out_shape=jax.ShapeDtypeStruct((nch * E, CH), jnp.float32),
    )(edge_attr, W, b.reshape(nch, CH))


def _sc_gine_agg(src, dst, c_all, x_chunks, nch, N, E):
    ept = E // NSUB
    nb = ept // KB
    zb = N // ZR
    mesh = plsc.VectorSubcoreMesh(core_axis_name="c", subcore_axis_name="s")
    out_type = tuple(jax.ShapeDtypeStruct((N, CH), jnp.float32)
                     for _ in range(nch))
    f32, i32 = jnp.float32, jnp.int32
    scratch = [pltpu.VMEM((KB,), i32), pltpu.VMEM((KB,), i32),
               pltpu.VMEM((KB,), i32), pltpu.VMEM((KB,), i32),
               pltpu.VMEM((KB, CH), f32), pltpu.VMEM((KB, CH), f32),
               pltpu.VMEM((KB, CH), f32), pltpu.VMEM((KB, CH), f32),
               pltpu.VMEM((ZR, CH), f32),
               pltpu.VMEM_SHARED((N, CH), f32),
               pltpu.SemaphoreType.DMA, pltpu.SemaphoreType.DMA,
               pltpu.SemaphoreType.DMA, pltpu.SemaphoreType.DMA,
               pltpu.SemaphoreType.DMA, pltpu.SemaphoreType.DMA,
               pltpu.SemaphoreType.DMA, pltpu.SemaphoreType.DMA]

    @functools.partial(pl.kernel, out_type=out_type, mesh=mesh,
                       scratch_types=scratch)
    def k(src_h, dst_h, c_h, *rest):
        xs = rest[:nch]
        outs = rest[nch:2 * nch]
        (ia0, ia1, dv0, dv1, g0, g1, c0, c1, zbuf, shared,
         sg0, sg1, si0, si1, sd0, sd1, ss0, ss1) = rest[2 * nch:]
        ia = (ia0, ia1)
        dv = (dv0, dv1)
        gb = (g0, g1)
        cb = (c0, c1)
        sg = (sg0, sg1)
        si = (si0, si1)
        sd = (sd0, sd1)
        ss = (ss0, ss1)
        cid = lax.axis_index("c")
        sid = lax.axis_index("s")

        def zrow(r, carry):
            for q in range(CH // 16):
                zbuf[r, pl.ds(q * 16, 16)] = jnp.zeros((16,), f32)
            return carry
        lax.fori_loop(0, ZR, zrow, 0)

        def issue_io(b, p, chunk):
            e0 = pl.multiple_of(sid * ept + b * KB, KB)
            eh = pl.multiple_of(chunk * E + e0, KB)
            pltpu.async_copy(src_h.at[pl.ds(e0, KB)], ia[p], si[p])
            pltpu.async_copy(c_h.at[pl.ds(eh, KB), :], cb[p], si[p])

        def drain_io(p):
            pltpu.make_async_copy(src_h.at[pl.ds(0, KB)], ia[p],
                                  si[p]).wait()
            pltpu.make_async_copy(c_h.at[pl.ds(0, KB), :], cb[p],
                                  si[p]).wait()

        def issue_dv(b, p):
            e0 = pl.multiple_of(sid * ept + b * KB, KB)
            pltpu.async_copy(dst_h.at[pl.ds(e0, KB)], dv[p], sd[p])

        def drain_dv(p):
            pltpu.make_async_copy(dst_h.at[pl.ds(0, KB)], dv[p],
                                  sd[p]).wait()

        def issue_gather(p, chunk):
            pltpu.async_copy(xs[chunk].at[ia[p]], gb[p], sg[p])

        def drain_gather(p, chunk):
            pltpu.make_async_copy(xs[chunk].at[ia[p]], gb[p],
                                  sg[p]).wait()

        for chunk in range(nch):
            @pl.when(cid == (chunk % NCORE))
            def _(chunk=chunk):
                nz_full, rem = divmod(zb, NSUB)
                for z in range(nz_full):
                    blk = sid + NSUB * z
                    pltpu.async_copy(zbuf, shared.at[pl.ds(blk * ZR, ZR), :],
                                     ss0)
                if rem:
                    @pl.when(sid < rem)
                    def _():
                        blk = sid + NSUB * nz_full
                        pltpu.async_copy(
                            zbuf, shared.at[pl.ds(blk * ZR, ZR), :], ss0)
                for z in range(nz_full):
                    blk = sid + NSUB * z
                    pltpu.make_async_copy(
                        zbuf, shared.at[pl.ds(blk * ZR, ZR), :], ss0).wait()
                if rem:
                    @pl.when(sid < rem)
                    def _():
                        blk = sid + NSUB * nz_full
                        pltpu.make_async_copy(
                            zbuf, shared.at[pl.ds(blk * ZR, ZR), :],
                            ss0).wait()
                plsc.subcore_barrier()

                e0 = pl.multiple_of(sid * ept, KB)
                eh = pl.multiple_of(chunk * E + e0, KB)
                pltpu.sync_copy(src_h.at[pl.ds(e0, KB)], ia[0])
                pltpu.sync_copy(c_h.at[pl.ds(eh, KB), :], cb[0])
                issue_dv(0, 0)
                issue_gather(0, chunk)
                issue_io(1, 1, chunk)

                def scat_desc(p):
                    return pltpu.make_async_copy(gb[p], shared.at[dv[p]],
                                                 ss[p])

                def step(b, p):
                    @pl.when(b + 1 <= nb - 1)
                    def _():
                        drain_io(1 - p)
                    drain_gather(p, chunk)

                    @plsc.parallel_loop(0, KB, 1, unroll=4)
                    def _(r):
                        for q in range(CH // 16):
                            s = pl.ds(q * 16, 16)
                            gb[p][r, s] = jnp.maximum(
                                gb[p][r, s] + cb[p][r, s], 0.0)

                    @pl.when(b + 2 <= nb - 1)
                    def _():
                        issue_io(b + 2, p, chunk)
                    @pl.when(b >= 1)
                    def _():
                        scat_desc(1 - p).wait()
                    @pl.when(b + 1 <= nb - 1)
                    def _():
                        issue_gather(1 - p, chunk)
                        issue_dv(b + 1, 1 - p)
                    drain_dv(p)
                    scat_desc(p).start(add=True)

                def pair(j, carry):
                    step(2 * j, 0)
                    step(2 * j + 1, 1)
                    return carry
                lax.fori_loop(0, nb // 2, pair, 0)
                if nb % 2:
                    step(nb - 1, 0)
                scat_desc((nb - 1) % 2).wait()
                plsc.subcore_barrier()

                nz_full, rem = divmod(zb, NSUB)
                for z in range(nz_full):
                    blk = sid + NSUB * z
                    pltpu.async_copy(shared.at[pl.ds(blk * ZR, ZR), :],
                                     outs[chunk].at[pl.ds(blk * ZR, ZR), :],
                                     ss0)
                if rem:
                    @pl.when(sid < rem)
                    def _():
                        blk = sid + NSUB * nz_full
                        pltpu.async_copy(
                            shared.at[pl.ds(blk * ZR, ZR), :],
                            outs[chunk].at[pl.ds(blk * ZR, ZR), :], ss0)
                for z in range(nz_full):
                    blk = sid + NSUB * z
                    pltpu.make_async_copy(
                        shared.at[pl.ds(blk * ZR, ZR), :],
                        outs[chunk].at[pl.ds(blk * ZR, ZR), :], ss0).wait()
                if rem:
                    @pl.when(sid < rem)
                    def _():
                        blk = sid + NSUB * nz_full
                        pltpu.make_async_copy(
                            shared.at[pl.ds(blk * ZR, ZR), :],
                            outs[chunk].at[pl.ds(blk * ZR, ZR), :],
                            ss0).wait()

    return k(src, dst, c_all, *x_chunks)


def _gin_mlp(self_chunks, agg_chunks, W1, b1, W2, b2, relu_out, out_chunked):
    N = self_chunks[0].shape[0]
    nin = len(self_chunks)
    Hh = W1.shape[0]
    nbl = 10
    Nb = N // nbl
    nco = Hh // CH

    def body(*refs):
        ins = refs[:nin]
        aggs = refs[nin:2 * nin]
        w1, bb1, w2, bb2 = refs[2 * nin:2 * nin + 4]
        outs = refs[2 * nin + 4:]
        h0 = jnp.concatenate(
            [ins[i][...] + aggs[i][...] for i in range(nin)], axis=1)
        t = _lrelu(_dot_t(h0, w1[...]) + bb1[...])
        u = _lrelu(_dot_t(t, w2[...]) + bb2[...])
        if relu_out:
            u = jnp.maximum(u, 0.0)
        if out_chunked:
            for q in range(nco):
                outs[q][...] = u[:, q * CH:(q + 1) * CH]
        else:
            outs[0][...] = u

    in_specs = ([pl.BlockSpec((Nb, CH), lambda i: (i, 0))] * (2 * nin)
                + [pl.BlockSpec(W1.shape, lambda i: (0, 0)),
                   pl.BlockSpec((1, Hh), lambda i: (0, 0)),
                   pl.BlockSpec(W2.shape, lambda i: (0, 0)),
                   pl.BlockSpec((1, Hh), lambda i: (0, 0))])
    if out_chunked:
        out_shape = tuple(jax.ShapeDtypeStruct((N, CH), jnp.float32)
                          for _ in range(nco))
        out_specs = tuple(pl.BlockSpec((Nb, CH), lambda i: (i, 0))
                          for _ in range(nco))
    else:
        out_shape = jax.ShapeDtypeStruct((N, Hh), jnp.float32)
        out_specs = pl.BlockSpec((Nb, Hh), lambda i: (i, 0))
    return pl.pallas_call(
        body, grid=(nbl,), in_specs=in_specs, out_specs=out_specs,
        out_shape=out_shape,
    )(*self_chunks, *agg_chunks, W1, b1.reshape(1, Hh), W2,
      b2.reshape(1, Hh))


def _head(h2, target_object, Wa1, ba1, Wa2, ba2):
    Bb, S = target_object.shape
    Hh = Wa1.shape[0]
    A = Wa2.shape[0]

    def body(to, hb, w1, bb1, w2, bb2, out):
        h3 = hb[...].reshape(Bb, S, Hh)
        sel = lax.dot_general(to[...], h3, (((1,), (1,)), ((0,), (0,))),
                              preferred_element_type=jnp.float32)
        a = _lrelu(_dot_t(sel, w1[...]) + bb1[...])
        a = _lrelu(_dot_t(a, w2[...]) + bb2[...])
        m = jnp.max(a, axis=1, keepdims=True)
        e = jnp.exp(a - m)
        out[...] = e / jnp.sum(e, axis=1, keepdims=True)

    return pl.pallas_call(
        body, grid=(1,),
        in_specs=[pl.BlockSpec((Bb, S), lambda i: (0, 0)),
                  pl.BlockSpec((Bb * S, Hh), lambda i: (0, 0)),
                  pl.BlockSpec((Hh, Hh), lambda i: (0, 0)),
                  pl.BlockSpec((1, Hh), lambda i: (0, 0)),
                  pl.BlockSpec((A, Hh), lambda i: (0, 0)),
                  pl.BlockSpec((1, A), lambda i: (0, 0))],
        out_specs=pl.BlockSpec((Bb, A), lambda i: (0, 0)),
        out_shape=jax.ShapeDtypeStruct((Bb, A), jnp.float32),
    )(target_object, h2, Wa1, ba1.reshape(1, Hh), Wa2, ba2.reshape(1, A))


def kernel(x, edge_index, edge_attr, target_object, W_e1, b_e1, W11, b11,
           W12, b12, W_e2, b_e2, W21, b21, W22, b22, Wa1, ba1, Wa2, ba2):
    N, D = x.shape
    E = edge_attr.shape[0]
    Hh = W11.shape[0]
    src = edge_index[0]
    dst = edge_index[1]

    x_chunks = [x[:, i * CH:(i + 1) * CH] for i in range(D // CH)]

    c1 = _edge_bias(edge_attr, W_e1, b_e1, D // CH)

    agg1 = _sc_gine_agg(src, dst, c1, x_chunks, D // CH, N, E)
    c2 = _edge_bias(edge_attr, W_e2, b_e2, Hh // CH)
    h1 = _gin_mlp(x_chunks, list(agg1), W11, b11, W12, b12,
                  relu_out=True, out_chunked=True)
    agg2 = _sc_gine_agg(src, dst, c2, list(h1), Hh // CH, N, E)
    h2 = _gin_mlp(list(h1), list(agg2), W21, b21, W22, b22,
                  relu_out=False, out_chunked=False)
    return _head(h2, target_object, Wa1, ba1, Wa2, ba2)
